# Initial kernel scaffold; baseline (speedup 1.0000x reference)
import jax
import jax.numpy as jnp
from jax.experimental import pallas as pl


def kernel(x, edge_index_connections, edge_index_destinations, edge_index_trains, conv1_Wl, conv1_bl, conv1_Wr, conv2_Wl, conv2_bl, conv2_Wr, conv3_Wl, conv3_bl, conv3_Wr, conv4_Wl, conv4_bl, conv4_Wr, conv5_Wl, conv5_bl, conv5_Wr, lin1_W, lin1_b, lin2_W, lin2_b):
    raise NotImplementedError("write your pallas kernel here")


# speedup vs baseline: 1.9827x; 1.9827x over previous
import functools
import math

import jax
import jax.numpy as jnp
from jax import lax
from jax.experimental import pallas as pl
from jax.experimental.pallas import tpu as pltpu
from jax.experimental.pallas import tpu_sc as plsc

_N = 10000
_D = 128
_NC = 2
_NS = 16
_LANES = 128
_ACC_ROWS = 10112
_ZROWS = _ACC_ROWS // _NS
_OROWS = _ZROWS


_KB = 8


def _prep_edges(ei):
    e = ei.shape[1]
    c = _KB * math.ceil(e / (_NC * _NS * _LANES * _KB))
    ep = c * _NC * _NS * _LANES
    pad = ep - e
    src = jnp.concatenate([ei[0], jnp.zeros((pad,), jnp.int32)])
    dst = jnp.concatenate([ei[1], jnp.full((pad,), _N, jnp.int32)])
    return src.reshape(_NC * _NS, c, _LANES), dst.reshape(_NC * _NS, c, _LANES)


def _sc_segsum(x, src3, dst3, zrows):
    c = src3.shape[1]

    @functools.partial(
        pl.kernel,
        out_type=jax.ShapeDtypeStruct((_NC, _ACC_ROWS, _D), jnp.float32),
        mesh=plsc.VectorSubcoreMesh(core_axis_name="c", subcore_axis_name="s"),
        scratch_types=[
            pltpu.VMEM((_KB, _LANES), jnp.int32),
            pltpu.VMEM((_KB, _LANES), jnp.int32),
            pltpu.VMEM((_LANES, _D), jnp.float32),
            pltpu.VMEM_SHARED((_ACC_ROWS, _D), jnp.float32),
            pltpu.SemaphoreType.DMA,
        ],
    )
    def run(x_hbm, src_hbm, dst_hbm, z_hbm, out_hbm, src_v, dst_v, rows_v, acc, sem):
        cid = lax.axis_index("c")
        sid = lax.axis_index("s")
        wid = cid * _NS + sid
        pltpu.sync_copy(z_hbm, acc.at[pl.ds(sid * _ZROWS, _ZROWS)])
        plsc.subcore_barrier()

        def body(b, carry):
            off = pl.multiple_of(b * _KB, _KB)
            pltpu.sync_copy(src_hbm.at[wid, pl.ds(off, _KB)], src_v)
            pltpu.sync_copy(dst_hbm.at[wid, pl.ds(off, _KB)], dst_v)
            for j in range(_KB):
                pltpu.async_copy(x_hbm.at[src_v.at[j]], rows_v, sem).wait()
                pltpu.sync_copy(rows_v, acc.at[dst_v.at[j]], add=True)
            return carry

        lax.fori_loop(0, c // _KB, body, 0)
        plsc.subcore_barrier()
        pltpu.sync_copy(
            acc.at[pl.ds(sid * _OROWS, _OROWS)],
            out_hbm.at[cid, pl.ds(sid * _OROWS, _OROWS)],
        )

    return run(x, src3, dst3, zrows)


_RB = 1000


def _dense_body(s0, s1, c0, c1, x, wl, bl, wr, o, *, relu):
    cnt = c0[:, 0:1] + c1[:, 0:1]
    inv = 1.0 / jnp.maximum(cnt, 1.0)
    mean = (s0[...] + s1[...]) * inv
    y = (jnp.dot(mean, wl[...], preferred_element_type=jnp.float32)
         + bl[...]
         + jnp.dot(x[...], wr[...], preferred_element_type=jnp.float32))
    o[...] = jnp.maximum(y, 0.0) if relu else y


def _dense(s0, s1, c0, c1, x, wl_t, bl, wr_t, relu):
    w = c0.shape[1]
    return pl.pallas_call(
        functools.partial(_dense_body, relu=relu),
        grid=(_N // _RB,),
        in_specs=[
            pl.BlockSpec((_RB, _D), lambda i: (i, 0)),
            pl.BlockSpec((_RB, _D), lambda i: (i, 0)),
            pl.BlockSpec((_RB, w), lambda i: (i, 0)),
            pl.BlockSpec((_RB, w), lambda i: (i, 0)),
            pl.BlockSpec((_RB, _D), lambda i: (i, 0)),
            pl.BlockSpec((_D, _D), lambda i: (0, 0)),
            pl.BlockSpec((_D,), lambda i: (0,)),
            pl.BlockSpec((_D, _D), lambda i: (0, 0)),
        ],
        out_specs=pl.BlockSpec((_RB, _D), lambda i: (i, 0)),
        out_shape=jax.ShapeDtypeStruct((_N, _D), jnp.float32),
    )(s0, s1, c0, c1, x, wl_t, bl, wr_t)


def _linear2_body(x, w1, b1, w2, b2, o):
    h = jnp.dot(x[...], w1[...], preferred_element_type=jnp.float32) + b1[...]
    o[...] = jnp.dot(h, w2[...], preferred_element_type=jnp.float32) + b2[...]


def _linear2(x, w1_t, b1, w2_t, b2):
    return pl.pallas_call(
        _linear2_body,
        grid=(_N // _RB,),
        in_specs=[
            pl.BlockSpec((_RB, _D), lambda i: (i, 0)),
            pl.BlockSpec((_D, _D), lambda i: (0, 0)),
            pl.BlockSpec((_D,), lambda i: (0,)),
            pl.BlockSpec((_D, _D), lambda i: (0, 0)),
            pl.BlockSpec((_D,), lambda i: (0,)),
        ],
        out_specs=pl.BlockSpec((_RB, _D), lambda i: (i, 0)),
        out_shape=jax.ShapeDtypeStruct((_N, _D), jnp.float32),
    )(x, w1_t, b1, w2_t, b2)


def kernel(x, edge_index_connections, edge_index_destinations, edge_index_trains,
           conv1_Wl, conv1_bl, conv1_Wr,
           conv2_Wl, conv2_bl, conv2_Wr,
           conv3_Wl, conv3_bl, conv3_Wr,
           conv4_Wl, conv4_bl, conv4_Wr,
           conv5_Wl, conv5_bl, conv5_Wr,
           lin1_W, lin1_b, lin2_W, lin2_b):
    zrows = jnp.zeros((_ZROWS, _D), jnp.float32)
    ones_x = jnp.ones((_N, _D), jnp.float32)

    con = _prep_edges(edge_index_connections)
    dest = _prep_edges(edge_index_destinations)
    trn = _prep_edges(edge_index_trains)

    cnt_con = _sc_segsum(ones_x, con[0], con[1], zrows)
    cnt_dest = _sc_segsum(ones_x, dest[0], dest[1], zrows)
    cnt_trn = _sc_segsum(ones_x, trn[0], trn[1], zrows)

    layers = [
        (con, cnt_con, conv1_Wl, conv1_bl, conv1_Wr, True),
        (trn, cnt_trn, conv2_Wl, conv2_bl, conv2_Wr, False),
        (con, cnt_con, conv3_Wl, conv3_bl, conv3_Wr, True),
        (con, cnt_con, conv3_Wl, conv3_bl, conv3_Wr, True),
        (dest, cnt_dest, conv4_Wl, conv4_bl, conv4_Wr, True),
        (con, cnt_con, conv5_Wl, conv5_bl, conv5_Wr, True),
        (con, cnt_con, conv5_Wl, conv5_bl, conv5_Wr, True),
    ]

    h = x
    for (src3, dst3), cnt, wl, bl, wr, relu in layers:
        p = _sc_segsum(h, src3, dst3, zrows)
        h = _dense(p[0], p[1], cnt[0], cnt[1], h, wl.T, bl, wr.T, relu)

    return _linear2(h, lin1_W.T, lin1_b, lin2_W.T, lin2_b)


# speedup vs baseline: 2.1171x; 1.0678x over previous
import functools
import math

import jax
import jax.numpy as jnp
from jax import lax
from jax.experimental import pallas as pl
from jax.experimental.pallas import tpu as pltpu
from jax.experimental.pallas import tpu_sc as plsc

_N = 10000
_D = 128
_NC = 2
_NS = 16
_LANES = 128
_ACC_ROWS = 10112
_ZROWS = _ACC_ROWS // _NS
_OROWS = _ZROWS


_KB = 16


def _prep_edges(ei):
    e = ei.shape[1]
    c = _KB * math.ceil(e / (_NC * _NS * _LANES * _KB))
    ep = c * _NC * _NS * _LANES
    pad = ep - e
    src = jnp.concatenate([ei[0], jnp.zeros((pad,), jnp.int32)])
    dst = jnp.concatenate([ei[1], jnp.full((pad,), _N, jnp.int32)])
    return src.reshape(_NC * _NS, c, _LANES), dst.reshape(_NC * _NS, c, _LANES)


def _sc_segsum(x, src3, dst3, zrows):
    c = src3.shape[1]

    @functools.partial(
        pl.kernel,
        out_type=jax.ShapeDtypeStruct((_NC, _ACC_ROWS, _D), jnp.float32),
        mesh=plsc.VectorSubcoreMesh(core_axis_name="c", subcore_axis_name="s"),
        scratch_types=[
            pltpu.VMEM((_KB, _LANES), jnp.int32),
            pltpu.VMEM((_KB, _LANES), jnp.int32),
            pltpu.VMEM((2, _LANES, _D), jnp.float32),
            pltpu.VMEM_SHARED((_ACC_ROWS, _D), jnp.float32),
            [pltpu.SemaphoreType.DMA] * 2,
            [pltpu.SemaphoreType.DMA] * 2,
        ],
    )
    def run(x_hbm, src_hbm, dst_hbm, z_hbm, out_hbm, src_v, dst_v, rows_v, acc,
            gsem, ssem):
        cid = lax.axis_index("c")
        sid = lax.axis_index("s")
        wid = cid * _NS + sid
        pltpu.sync_copy(z_hbm, acc.at[pl.ds(sid * _ZROWS, _ZROWS)])
        plsc.subcore_barrier()

        def body(b, carry):
            off = pl.multiple_of(b * _KB, _KB)
            pltpu.sync_copy(src_hbm.at[wid, pl.ds(off, _KB)], src_v)
            pltpu.sync_copy(dst_hbm.at[wid, pl.ds(off, _KB)], dst_v)
            gathers = [None, None]
            scatters = [None, None]
            gathers[0] = pltpu.async_copy(
                x_hbm.at[src_v.at[0]], rows_v.at[0], gsem[0])
            for j in range(_KB):
                p = j % 2
                gathers[p].wait()
                scatters[p] = pltpu.async_copy(
                    rows_v.at[p], acc.at[dst_v.at[j]], ssem[p], add=True)
                if j + 1 < _KB:
                    q = (j + 1) % 2
                    if scatters[q] is not None:
                        scatters[q].wait()
                    gathers[q] = pltpu.async_copy(
                        x_hbm.at[src_v.at[j + 1]], rows_v.at[q], gsem[q])
            scatters[0].wait()
            scatters[1].wait()
            return carry

        lax.fori_loop(0, c // _KB, body, 0)
        plsc.subcore_barrier()
        pltpu.sync_copy(
            acc.at[pl.ds(sid * _OROWS, _OROWS)],
            out_hbm.at[cid, pl.ds(sid * _OROWS, _OROWS)],
        )

    return run(x, src3, dst3, zrows)


_RB = 1000


def _dense_body(s0, s1, c0, c1, x, wl, bl, wr, o, *, relu):
    cnt = c0[:, 0:1] + c1[:, 0:1]
    inv = 1.0 / jnp.maximum(cnt, 1.0)
    mean = (s0[...] + s1[...]) * inv
    y = (jnp.dot(mean, wl[...], preferred_element_type=jnp.float32)
         + bl[...]
         + jnp.dot(x[...], wr[...], preferred_element_type=jnp.float32))
    o[...] = jnp.maximum(y, 0.0) if relu else y


def _dense(s0, s1, c0, c1, x, wl_t, bl, wr_t, relu):
    w = c0.shape[1]
    return pl.pallas_call(
        functools.partial(_dense_body, relu=relu),
        grid=(_N // _RB,),
        in_specs=[
            pl.BlockSpec((_RB, _D), lambda i: (i, 0)),
            pl.BlockSpec((_RB, _D), lambda i: (i, 0)),
            pl.BlockSpec((_RB, w), lambda i: (i, 0)),
            pl.BlockSpec((_RB, w), lambda i: (i, 0)),
            pl.BlockSpec((_RB, _D), lambda i: (i, 0)),
            pl.BlockSpec((_D, _D), lambda i: (0, 0)),
            pl.BlockSpec((_D,), lambda i: (0,)),
            pl.BlockSpec((_D, _D), lambda i: (0, 0)),
        ],
        out_specs=pl.BlockSpec((_RB, _D), lambda i: (i, 0)),
        out_shape=jax.ShapeDtypeStruct((_N, _D), jnp.float32),
    )(s0, s1, c0, c1, x, wl_t, bl, wr_t)


def _linear2_body(x, w1, b1, w2, b2, o):
    h = jnp.dot(x[...], w1[...], preferred_element_type=jnp.float32) + b1[...]
    o[...] = jnp.dot(h, w2[...], preferred_element_type=jnp.float32) + b2[...]


def _linear2(x, w1_t, b1, w2_t, b2):
    return pl.pallas_call(
        _linear2_body,
        grid=(_N // _RB,),
        in_specs=[
            pl.BlockSpec((_RB, _D), lambda i: (i, 0)),
            pl.BlockSpec((_D, _D), lambda i: (0, 0)),
            pl.BlockSpec((_D,), lambda i: (0,)),
            pl.BlockSpec((_D, _D), lambda i: (0, 0)),
            pl.BlockSpec((_D,), lambda i: (0,)),
        ],
        out_specs=pl.BlockSpec((_RB, _D), lambda i: (i, 0)),
        out_shape=jax.ShapeDtypeStruct((_N, _D), jnp.float32),
    )(x, w1_t, b1, w2_t, b2)


def kernel(x, edge_index_connections, edge_index_destinations, edge_index_trains,
           conv1_Wl, conv1_bl, conv1_Wr,
           conv2_Wl, conv2_bl, conv2_Wr,
           conv3_Wl, conv3_bl, conv3_Wr,
           conv4_Wl, conv4_bl, conv4_Wr,
           conv5_Wl, conv5_bl, conv5_Wr,
           lin1_W, lin1_b, lin2_W, lin2_b):
    zrows = jnp.zeros((_ZROWS, _D), jnp.float32)
    ones_x = jnp.ones((_N, _D), jnp.float32)

    con = _prep_edges(edge_index_connections)
    dest = _prep_edges(edge_index_destinations)
    trn = _prep_edges(edge_index_trains)

    cnt_con = _sc_segsum(ones_x, con[0], con[1], zrows)
    cnt_dest = _sc_segsum(ones_x, dest[0], dest[1], zrows)
    cnt_trn = _sc_segsum(ones_x, trn[0], trn[1], zrows)

    layers = [
        (con, cnt_con, conv1_Wl, conv1_bl, conv1_Wr, True),
        (trn, cnt_trn, conv2_Wl, conv2_bl, conv2_Wr, False),
        (con, cnt_con, conv3_Wl, conv3_bl, conv3_Wr, True),
        (con, cnt_con, conv3_Wl, conv3_bl, conv3_Wr, True),
        (dest, cnt_dest, conv4_Wl, conv4_bl, conv4_Wr, True),
        (con, cnt_con, conv5_Wl, conv5_bl, conv5_Wr, True),
        (con, cnt_con, conv5_Wl, conv5_bl, conv5_Wr, True),
    ]

    h = x
    for (src3, dst3), cnt, wl, bl, wr, relu in layers:
        p = _sc_segsum(h, src3, dst3, zrows)
        h = _dense(p[0], p[1], cnt[0], cnt[1], h, wl.T, bl, wr.T, relu)

    return _linear2(h, lin1_W.T, lin1_b, lin2_W.T, lin2_b)


# speedup vs baseline: 2.1470x; 1.0141x over previous
import functools
import math

import jax
import jax.numpy as jnp
from jax import lax
from jax.experimental import pallas as pl
from jax.experimental.pallas import tpu as pltpu
from jax.experimental.pallas import tpu_sc as plsc

_N = 10000
_D = 128
_NC = 2
_NS = 16
_LANES = 128
_ACC_ROWS = 10112
_ZROWS = _ACC_ROWS // _NS
_OROWS = _ZROWS


_KB = 16


def _retile(a):
    return pl.pallas_call(
        lambda x_ref, o_ref: o_ref.__setitem__((...,), x_ref[...]),
        out_shape=jax.ShapeDtypeStruct(a.shape, a.dtype),
    )(a)


def _prep_edges(ei):
    e = ei.shape[1]
    c = _KB * math.ceil(e / (_NC * _NS * _LANES * _KB))
    ep = c * _NC * _NS * _LANES
    pad = ep - e
    src = jnp.concatenate([ei[0], jnp.zeros((pad,), jnp.int32)])
    dst = jnp.concatenate([ei[1], jnp.full((pad,), _N, jnp.int32)])
    return (_retile(src.reshape(_NC * _NS, c, _LANES)),
            _retile(dst.reshape(_NC * _NS, c, _LANES)))


def _sc_segsum(x, src3, dst3, zrows):
    c = src3.shape[1]

    @functools.partial(
        pl.kernel,
        out_type=jax.ShapeDtypeStruct((_NC, _ACC_ROWS, _D), jnp.float32),
        mesh=plsc.VectorSubcoreMesh(core_axis_name="c", subcore_axis_name="s"),
        scratch_types=[
            pltpu.VMEM((_KB, _LANES), jnp.int32),
            pltpu.VMEM((_KB, _LANES), jnp.int32),
            pltpu.VMEM((2, _LANES, _D), jnp.float32),
            pltpu.VMEM_SHARED((_ACC_ROWS, _D), jnp.float32),
            [pltpu.SemaphoreType.DMA] * 2,
            [pltpu.SemaphoreType.DMA] * 2,
        ],
    )
    def run(x_hbm, src_hbm, dst_hbm, z_hbm, out_hbm, src_v, dst_v, rows_v, acc,
            gsem, ssem):
        cid = lax.axis_index("c")
        sid = lax.axis_index("s")
        wid = cid * _NS + sid
        pltpu.sync_copy(z_hbm, acc.at[pl.ds(sid * _ZROWS, _ZROWS)])
        plsc.subcore_barrier()

        def body(b, carry):
            off = pl.multiple_of(b * _KB, _KB)
            pltpu.sync_copy(src_hbm.at[wid, pl.ds(off, _KB)], src_v)
            pltpu.sync_copy(dst_hbm.at[wid, pl.ds(off, _KB)], dst_v)
            gathers = [None, None]
            scatters = [None, None]
            gathers[0] = pltpu.async_copy(
                x_hbm.at[src_v.at[0]], rows_v.at[0], gsem[0])
            for j in range(_KB):
                p = j % 2
                gathers[p].wait()
                scatters[p] = pltpu.async_copy(
                    rows_v.at[p], acc.at[dst_v.at[j]], ssem[p], add=True)
                if j + 1 < _KB:
                    q = (j + 1) % 2
                    if scatters[q] is not None:
                        scatters[q].wait()
                    gathers[q] = pltpu.async_copy(
                        x_hbm.at[src_v.at[j + 1]], rows_v.at[q], gsem[q])
            scatters[0].wait()
            scatters[1].wait()
            return carry

        lax.fori_loop(0, c // _KB, body, 0)
        plsc.subcore_barrier()
        pltpu.sync_copy(
            acc.at[pl.ds(sid * _OROWS, _OROWS)],
            out_hbm.at[cid, pl.ds(sid * _OROWS, _OROWS)],
        )

    return run(x, src3, dst3, zrows)


_RB = 1000


def _dense_body(s0, s1, c0, c1, x, wl, bl, wr, o, *, relu):
    cnt = c0[:, 0:1] + c1[:, 0:1]
    inv = 1.0 / jnp.maximum(cnt, 1.0)
    mean = (s0[...] + s1[...]) * inv
    y = (jnp.dot(mean, wl[...], preferred_element_type=jnp.float32)
         + bl[...]
         + jnp.dot(x[...], wr[...], preferred_element_type=jnp.float32))
    o[...] = jnp.maximum(y, 0.0) if relu else y


def _dense(s0, s1, c0, c1, x, wl_t, bl, wr_t, relu):
    w = c0.shape[1]
    return pl.pallas_call(
        functools.partial(_dense_body, relu=relu),
        grid=(_N // _RB,),
        in_specs=[
            pl.BlockSpec((_RB, _D), lambda i: (i, 0)),
            pl.BlockSpec((_RB, _D), lambda i: (i, 0)),
            pl.BlockSpec((_RB, w), lambda i: (i, 0)),
            pl.BlockSpec((_RB, w), lambda i: (i, 0)),
            pl.BlockSpec((_RB, _D), lambda i: (i, 0)),
            pl.BlockSpec((_D, _D), lambda i: (0, 0)),
            pl.BlockSpec((_D,), lambda i: (0,)),
            pl.BlockSpec((_D, _D), lambda i: (0, 0)),
        ],
        out_specs=pl.BlockSpec((_RB, _D), lambda i: (i, 0)),
        out_shape=jax.ShapeDtypeStruct((_N, _D), jnp.float32),
    )(s0, s1, c0, c1, x, wl_t, bl, wr_t)


def _linear2_body(x, w1, b1, w2, b2, o):
    h = jnp.dot(x[...], w1[...], preferred_element_type=jnp.float32) + b1[...]
    o[...] = jnp.dot(h, w2[...], preferred_element_type=jnp.float32) + b2[...]


def _linear2(x, w1_t, b1, w2_t, b2):
    return pl.pallas_call(
        _linear2_body,
        grid=(_N // _RB,),
        in_specs=[
            pl.BlockSpec((_RB, _D), lambda i: (i, 0)),
            pl.BlockSpec((_D, _D), lambda i: (0, 0)),
            pl.BlockSpec((_D,), lambda i: (0,)),
            pl.BlockSpec((_D, _D), lambda i: (0, 0)),
            pl.BlockSpec((_D,), lambda i: (0,)),
        ],
        out_specs=pl.BlockSpec((_RB, _D), lambda i: (i, 0)),
        out_shape=jax.ShapeDtypeStruct((_N, _D), jnp.float32),
    )(x, w1_t, b1, w2_t, b2)


def kernel(x, edge_index_connections, edge_index_destinations, edge_index_trains,
           conv1_Wl, conv1_bl, conv1_Wr,
           conv2_Wl, conv2_bl, conv2_Wr,
           conv3_Wl, conv3_bl, conv3_Wr,
           conv4_Wl, conv4_bl, conv4_Wr,
           conv5_Wl, conv5_bl, conv5_Wr,
           lin1_W, lin1_b, lin2_W, lin2_b):
    zrows = _retile(jnp.zeros((_ZROWS, _D), jnp.float32))
    ones_x = _retile(jnp.ones((_N, _D), jnp.float32))

    con = _prep_edges(edge_index_connections)
    dest = _prep_edges(edge_index_destinations)
    trn = _prep_edges(edge_index_trains)

    cnt_con = _sc_segsum(ones_x, con[0], con[1], zrows)
    cnt_dest = _sc_segsum(ones_x, dest[0], dest[1], zrows)
    cnt_trn = _sc_segsum(ones_x, trn[0], trn[1], zrows)

    layers = [
        (con, cnt_con, conv1_Wl, conv1_bl, conv1_Wr, True),
        (trn, cnt_trn, conv2_Wl, conv2_bl, conv2_Wr, False),
        (con, cnt_con, conv3_Wl, conv3_bl, conv3_Wr, True),
        (con, cnt_con, conv3_Wl, conv3_bl, conv3_Wr, True),
        (dest, cnt_dest, conv4_Wl, conv4_bl, conv4_Wr, True),
        (con, cnt_con, conv5_Wl, conv5_bl, conv5_Wr, True),
        (con, cnt_con, conv5_Wl, conv5_bl, conv5_Wr, True),
    ]

    h = x
    for (src3, dst3), cnt, wl, bl, wr, relu in layers:
        p = _sc_segsum(h, src3, dst3, zrows)
        h = _dense(p[0], p[1], cnt[0], cnt[1], h, wl.T, bl, wr.T, relu)

    return _linear2(h, lin1_W.T, lin1_b, lin2_W.T, lin2_b)


# speedup vs baseline: 11.7384x; 5.4674x over previous
import functools
import math

import jax
import jax.numpy as jnp
from jax import lax
from jax.experimental import pallas as pl
from jax.experimental.pallas import tpu as pltpu
from jax.experimental.pallas import tpu_sc as plsc

_N = 10000
_D = 128
_NC = 2
_NS = 16
_LANES = 128
_ACC_ROWS = 10112
_ZROWS = _ACC_ROWS // _NS
_OROWS = _ZROWS


_KB = 16


def _prep_edges(ei):
    e = ei.shape[1]
    c = _KB * math.ceil(e / (_NC * _NS * _LANES * _KB))
    ep = c * _NC * _NS * _LANES
    pad = ep - e
    fill = (jnp.arange(pad, dtype=jnp.int32) * 37) % _N
    src = jnp.concatenate([ei[0], fill])
    dst = jnp.concatenate([ei[1], jnp.full((pad,), _N, jnp.int32)])
    return (src.reshape(_NC * _NS, c, _LANES),
            dst.reshape(_NC * _NS, c, _LANES))


def _sc_segsum(x, src3, dst3, zrows):
    c = src3.shape[1]

    @functools.partial(
        pl.kernel,
        out_type=jax.ShapeDtypeStruct((_NC, _ACC_ROWS, _D), jnp.float32),
        mesh=plsc.VectorSubcoreMesh(core_axis_name="c", subcore_axis_name="s"),
        scratch_types=[
            pltpu.VMEM((_KB, _LANES), jnp.int32),
            pltpu.VMEM((_KB, _LANES), jnp.int32),
            pltpu.VMEM((2, _LANES, _D), jnp.float32),
            pltpu.VMEM_SHARED((_ACC_ROWS, _D), jnp.float32),
            [pltpu.SemaphoreType.DMA] * 2,
            [pltpu.SemaphoreType.DMA] * 2,
        ],
    )
    def run(x_hbm, src_hbm, dst_hbm, z_hbm, out_hbm, src_v, dst_v, rows_v, acc,
            gsem, ssem):
        cid = lax.axis_index("c")
        sid = lax.axis_index("s")
        wid = cid * _NS + sid
        pltpu.sync_copy(z_hbm, acc.at[pl.ds(sid * _ZROWS, _ZROWS)])
        plsc.subcore_barrier()

        def body(b, carry):
            off = pl.multiple_of(b * _KB, _KB)
            pltpu.sync_copy(src_hbm.at[wid, pl.ds(off, _KB)], src_v)
            pltpu.sync_copy(dst_hbm.at[wid, pl.ds(off, _KB)], dst_v)
            gathers = [None, None]
            scatters = [None, None]
            gathers[0] = pltpu.async_copy(
                x_hbm.at[src_v.at[0]], rows_v.at[0], gsem[0])
            for j in range(_KB):
                p = j % 2
                gathers[p].wait()
                scatters[p] = pltpu.async_copy(
                    rows_v.at[p], acc.at[dst_v.at[j]], ssem[p], add=True)
                if j + 1 < _KB:
                    q = (j + 1) % 2
                    if scatters[q] is not None:
                        scatters[q].wait()
                    gathers[q] = pltpu.async_copy(
                        x_hbm.at[src_v.at[j + 1]], rows_v.at[q], gsem[q])
            scatters[0].wait()
            scatters[1].wait()
            return carry

        lax.fori_loop(0, c // _KB, body, 0)
        plsc.subcore_barrier()
        pltpu.sync_copy(
            acc.at[pl.ds(sid * _OROWS, _OROWS)],
            out_hbm.at[cid, pl.ds(sid * _OROWS, _OROWS)],
        )

    return run(x, src3, dst3, zrows)


_RB = 1000


def _dense_body(s0, s1, c0, c1, x, wl, bl, wr, o, *, relu):
    cnt = c0[:, 0:1] + c1[:, 0:1]
    inv = 1.0 / jnp.maximum(cnt, 1.0)
    mean = (s0[...] + s1[...]) * inv
    y = (jnp.dot(mean, wl[...], preferred_element_type=jnp.float32)
         + bl[...]
         + jnp.dot(x[...], wr[...], preferred_element_type=jnp.float32))
    o[...] = jnp.maximum(y, 0.0) if relu else y


def _dense(s0, s1, c0, c1, x, wl_t, bl, wr_t, relu):
    w = c0.shape[1]
    return pl.pallas_call(
        functools.partial(_dense_body, relu=relu),
        grid=(_N // _RB,),
        in_specs=[
            pl.BlockSpec((_RB, _D), lambda i: (i, 0)),
            pl.BlockSpec((_RB, _D), lambda i: (i, 0)),
            pl.BlockSpec((_RB, w), lambda i: (i, 0)),
            pl.BlockSpec((_RB, w), lambda i: (i, 0)),
            pl.BlockSpec((_RB, _D), lambda i: (i, 0)),
            pl.BlockSpec((_D, _D), lambda i: (0, 0)),
            pl.BlockSpec((_D,), lambda i: (0,)),
            pl.BlockSpec((_D, _D), lambda i: (0, 0)),
        ],
        out_specs=pl.BlockSpec((_RB, _D), lambda i: (i, 0)),
        out_shape=jax.ShapeDtypeStruct((_N, _D), jnp.float32),
    )(s0, s1, c0, c1, x, wl_t, bl, wr_t)


def _linear2_body(x, w1, b1, w2, b2, o):
    h = jnp.dot(x[...], w1[...], preferred_element_type=jnp.float32) + b1[...]
    o[...] = jnp.dot(h, w2[...], preferred_element_type=jnp.float32) + b2[...]


def _linear2(x, w1_t, b1, w2_t, b2):
    return pl.pallas_call(
        _linear2_body,
        grid=(_N // _RB,),
        in_specs=[
            pl.BlockSpec((_RB, _D), lambda i: (i, 0)),
            pl.BlockSpec((_D, _D), lambda i: (0, 0)),
            pl.BlockSpec((_D,), lambda i: (0,)),
            pl.BlockSpec((_D, _D), lambda i: (0, 0)),
            pl.BlockSpec((_D,), lambda i: (0,)),
        ],
        out_specs=pl.BlockSpec((_RB, _D), lambda i: (i, 0)),
        out_shape=jax.ShapeDtypeStruct((_N, _D), jnp.float32),
    )(x, w1_t, b1, w2_t, b2)


def kernel(x, edge_index_connections, edge_index_destinations, edge_index_trains,
           conv1_Wl, conv1_bl, conv1_Wr,
           conv2_Wl, conv2_bl, conv2_Wr,
           conv3_Wl, conv3_bl, conv3_Wr,
           conv4_Wl, conv4_bl, conv4_Wr,
           conv5_Wl, conv5_bl, conv5_Wr,
           lin1_W, lin1_b, lin2_W, lin2_b):
    zrows = jnp.zeros((_ZROWS, _D), jnp.float32)
    ones_x = jnp.ones((_N, _D), jnp.float32)

    con = _prep_edges(edge_index_connections)
    dest = _prep_edges(edge_index_destinations)
    trn = _prep_edges(edge_index_trains)

    cnt_con = _sc_segsum(ones_x, con[0], con[1], zrows)
    cnt_dest = _sc_segsum(ones_x, dest[0], dest[1], zrows)
    cnt_trn = _sc_segsum(ones_x, trn[0], trn[1], zrows)

    layers = [
        (con, cnt_con, conv1_Wl, conv1_bl, conv1_Wr, True),
        (trn, cnt_trn, conv2_Wl, conv2_bl, conv2_Wr, False),
        (con, cnt_con, conv3_Wl, conv3_bl, conv3_Wr, True),
        (con, cnt_con, conv3_Wl, conv3_bl, conv3_Wr, True),
        (dest, cnt_dest, conv4_Wl, conv4_bl, conv4_Wr, True),
        (con, cnt_con, conv5_Wl, conv5_bl, conv5_Wr, True),
        (con, cnt_con, conv5_Wl, conv5_bl, conv5_Wr, True),
    ]

    h = x
    for (src3, dst3), cnt, wl, bl, wr, relu in layers:
        p = _sc_segsum(h, src3, dst3, zrows)
        h = _dense(p[0], p[1], cnt[0], cnt[1], h, wl.T, bl, wr.T, relu)

    return _linear2(h, lin1_W.T, lin1_b, lin2_W.T, lin2_b)


# speedup vs baseline: 11.9215x; 1.0156x over previous
import functools
import math

import jax
import jax.numpy as jnp
from jax import lax
from jax.experimental import pallas as pl
from jax.experimental.pallas import tpu as pltpu
from jax.experimental.pallas import tpu_sc as plsc

_N = 10000
_D = 128
_NC = 2
_NS = 16
_CHUNK = 120
_KB = 8
_NBUF = 3
_ACC_ROWS = 10112
_ZROWS = _ACC_ROWS // _NS


def _prep_edges(ei):
    e = ei.shape[1]
    grp = _NC * _NS * _CHUNK * _KB
    c = _KB * math.ceil(e / grp)
    ep = c * _NC * _NS * _CHUNK
    pad = ep - e
    fill = (jnp.arange(pad, dtype=jnp.int32) * 37) % _N
    src = jnp.concatenate([ei[0], fill])
    dst = jnp.concatenate([ei[1], jnp.full((pad,), _N, jnp.int32)])
    return (src.reshape(_NC * _NS, c, _CHUNK),
            dst.reshape(_NC * _NS, c, _CHUNK))


def _sc_segsum(x, src3, dst3, zrows):
    c = src3.shape[1]

    @functools.partial(
        pl.kernel,
        out_type=jax.ShapeDtypeStruct((_NC, _ACC_ROWS, _D), jnp.float32),
        mesh=plsc.VectorSubcoreMesh(core_axis_name="c", subcore_axis_name="s"),
        scratch_types=[
            pltpu.VMEM((_KB, _CHUNK), jnp.int32),
            pltpu.VMEM((_KB, _CHUNK), jnp.int32),
            pltpu.VMEM((_NBUF, _CHUNK, _D), jnp.float32),
            pltpu.VMEM_SHARED((_ACC_ROWS, _D), jnp.float32),
            [pltpu.SemaphoreType.DMA] * _NBUF,
            [pltpu.SemaphoreType.DMA] * _NBUF,
        ],
    )
    def run(x_hbm, src_hbm, dst_hbm, z_hbm, out_hbm, src_v, dst_v, rows_v, acc,
            gsem, ssem):
        cid = lax.axis_index("c")
        sid = lax.axis_index("s")
        wid = cid * _NS + sid
        pltpu.sync_copy(z_hbm, acc.at[pl.ds(sid * _ZROWS, _ZROWS)])
        plsc.subcore_barrier()

        def body(b, carry):
            off = pl.multiple_of(b * _KB, 8)
            pltpu.sync_copy(src_hbm.at[wid, pl.ds(off, _KB)], src_v)
            pltpu.sync_copy(dst_hbm.at[wid, pl.ds(off, _KB)], dst_v)
            gathers = [None] * _NBUF
            scatters = [None] * _NBUF
            for j in range(_NBUF - 1):
                gathers[j] = pltpu.async_copy(
                    x_hbm.at[src_v.at[j]], rows_v.at[j], gsem[j])
            for j in range(_KB):
                p = j % _NBUF
                gathers[p].wait()
                scatters[p] = pltpu.async_copy(
                    rows_v.at[p], acc.at[dst_v.at[j]], ssem[p], add=True)
                nj = j + _NBUF - 1
                if nj < _KB:
                    q = nj % _NBUF
                    if scatters[q] is not None:
                        scatters[q].wait()
                    gathers[q] = pltpu.async_copy(
                        x_hbm.at[src_v.at[nj]], rows_v.at[q], gsem[q])
            for s_ in scatters:
                if s_ is not None:
                    s_.wait()
            return carry

        lax.fori_loop(0, c // _KB, body, 0)
        plsc.subcore_barrier()
        pltpu.sync_copy(
            acc.at[pl.ds(sid * _ZROWS, _ZROWS)],
            out_hbm.at[cid, pl.ds(sid * _ZROWS, _ZROWS)],
        )

    return run(x, src3, dst3, zrows)


_RB = 1000


def _self_body(x, wr, bl, o):
    o[...] = (jnp.dot(x[...], wr[...], preferred_element_type=jnp.float32)
              + bl[...])


def _self_term(x, wr_t, bl):
    return pl.pallas_call(
        _self_body,
        grid=(_N // _RB,),
        in_specs=[
            pl.BlockSpec((_RB, _D), lambda i: (i, 0)),
            pl.BlockSpec((_D, _D), lambda i: (0, 0)),
            pl.BlockSpec((_D,), lambda i: (0,)),
        ],
        out_specs=pl.BlockSpec((_RB, _D), lambda i: (i, 0)),
        out_shape=jax.ShapeDtypeStruct((_N, _D), jnp.float32),
    )(x, wr_t, bl)


def _combine_body(s0, s1, c0, c1, sf, wl, o, *, relu):
    cnt = c0[:, 0:1] + c1[:, 0:1]
    inv = 1.0 / jnp.maximum(cnt, 1.0)
    mean = (s0[...] + s1[...]) * inv
    y = jnp.dot(mean, wl[...], preferred_element_type=jnp.float32) + sf[...]
    o[...] = jnp.maximum(y, 0.0) if relu else y


def _combine(s0, s1, c0, c1, sf, wl_t, relu):
    return pl.pallas_call(
        functools.partial(_combine_body, relu=relu),
        grid=(_N // _RB,),
        in_specs=[
            pl.BlockSpec((_RB, _D), lambda i: (i, 0)),
            pl.BlockSpec((_RB, _D), lambda i: (i, 0)),
            pl.BlockSpec((_RB, _D), lambda i: (i, 0)),
            pl.BlockSpec((_RB, _D), lambda i: (i, 0)),
            pl.BlockSpec((_RB, _D), lambda i: (i, 0)),
            pl.BlockSpec((_D, _D), lambda i: (0, 0)),
        ],
        out_specs=pl.BlockSpec((_RB, _D), lambda i: (i, 0)),
        out_shape=jax.ShapeDtypeStruct((_N, _D), jnp.float32),
    )(s0, s1, c0, c1, sf, wl_t)


def _linear2_body(x, w1, b1, w2, b2, o):
    h = jnp.dot(x[...], w1[...], preferred_element_type=jnp.float32) + b1[...]
    o[...] = jnp.dot(h, w2[...], preferred_element_type=jnp.float32) + b2[...]


def _linear2(x, w1_t, b1, w2_t, b2):
    return pl.pallas_call(
        _linear2_body,
        grid=(_N // _RB,),
        in_specs=[
            pl.BlockSpec((_RB, _D), lambda i: (i, 0)),
            pl.BlockSpec((_D, _D), lambda i: (0, 0)),
            pl.BlockSpec((_D,), lambda i: (0,)),
            pl.BlockSpec((_D, _D), lambda i: (0, 0)),
            pl.BlockSpec((_D,), lambda i: (0,)),
        ],
        out_specs=pl.BlockSpec((_RB, _D), lambda i: (i, 0)),
        out_shape=jax.ShapeDtypeStruct((_N, _D), jnp.float32),
    )(x, w1_t, b1, w2_t, b2)


def kernel(x, edge_index_connections, edge_index_destinations, edge_index_trains,
           conv1_Wl, conv1_bl, conv1_Wr,
           conv2_Wl, conv2_bl, conv2_Wr,
           conv3_Wl, conv3_bl, conv3_Wr,
           conv4_Wl, conv4_bl, conv4_Wr,
           conv5_Wl, conv5_bl, conv5_Wr,
           lin1_W, lin1_b, lin2_W, lin2_b):
    zrows = jnp.zeros((_ZROWS, _D), jnp.float32)
    ones_x = jnp.ones((_N, _D), jnp.float32)

    con = _prep_edges(edge_index_connections)
    dest = _prep_edges(edge_index_destinations)
    trn = _prep_edges(edge_index_trains)

    cnt_con = _sc_segsum(ones_x, con[0], con[1], zrows)
    cnt_dest = _sc_segsum(ones_x, dest[0], dest[1], zrows)
    cnt_trn = _sc_segsum(ones_x, trn[0], trn[1], zrows)

    layers = [
        (con, cnt_con, conv1_Wl, conv1_bl, conv1_Wr, True),
        (trn, cnt_trn, conv2_Wl, conv2_bl, conv2_Wr, False),
        (con, cnt_con, conv3_Wl, conv3_bl, conv3_Wr, True),
        (con, cnt_con, conv3_Wl, conv3_bl, conv3_Wr, True),
        (dest, cnt_dest, conv4_Wl, conv4_bl, conv4_Wr, True),
        (con, cnt_con, conv5_Wl, conv5_bl, conv5_Wr, True),
        (con, cnt_con, conv5_Wl, conv5_bl, conv5_Wr, True),
    ]

    h = x
    for (src3, dst3), cnt, wl, bl, wr, relu in layers:
        p = _sc_segsum(h, src3, dst3, zrows)
        sf = _self_term(h, wr.T, bl)
        h = _combine(p[0], p[1], cnt[0], cnt[1], sf, wl.T, relu)

    return _linear2(h, lin1_W.T, lin1_b, lin2_W.T, lin2_b)


# speedup vs baseline: 12.5152x; 1.0498x over previous
import functools
import math

import jax
import jax.numpy as jnp
from jax import lax
from jax.experimental import pallas as pl
from jax.experimental.pallas import tpu as pltpu
from jax.experimental.pallas import tpu_sc as plsc

_N = 10000
_D = 128
_NC = 2
_NS = 16
_CHUNK = 120
_KB = 16
_NBUF = 3
_ACC_ROWS = 10112
_ZROWS = _ACC_ROWS // _NS


def _prep_edges(ei):
    e = ei.shape[1]
    grp = _NC * _NS * _CHUNK * _KB
    c = _KB * math.ceil(e / grp)
    ep = c * _NC * _NS * _CHUNK
    pad = ep - e
    fill = (jnp.arange(pad, dtype=jnp.int32) * 37) % _N
    src = jnp.concatenate([ei[0], fill])
    dst = jnp.concatenate([ei[1], jnp.full((pad,), _N, jnp.int32)])
    return (src.reshape(_NC * _NS, c, _CHUNK),
            dst.reshape(_NC * _NS, c, _CHUNK))


def _sc_segsum(x, src3, dst3, zrows):
    c = src3.shape[1]

    @functools.partial(
        pl.kernel,
        out_type=jax.ShapeDtypeStruct((_NC, _ACC_ROWS, _D), jnp.float32),
        mesh=plsc.VectorSubcoreMesh(core_axis_name="c", subcore_axis_name="s"),
        scratch_types=[
            pltpu.VMEM((_KB, _CHUNK), jnp.int32),
            pltpu.VMEM((_KB, _CHUNK), jnp.int32),
            pltpu.VMEM((_NBUF, _CHUNK, _D), jnp.float32),
            pltpu.VMEM_SHARED((_ACC_ROWS, _D), jnp.float32),
            [pltpu.SemaphoreType.DMA] * _NBUF,
            [pltpu.SemaphoreType.DMA] * _NBUF,
        ],
    )
    def run(x_hbm, src_hbm, dst_hbm, z_hbm, out_hbm, src_v, dst_v, rows_v, acc,
            gsem, ssem):
        cid = lax.axis_index("c")
        sid = lax.axis_index("s")
        wid = cid * _NS + sid
        pltpu.sync_copy(z_hbm, acc.at[pl.ds(sid * _ZROWS, _ZROWS)])
        plsc.subcore_barrier()

        def body(b, carry):
            off = pl.multiple_of(b * _KB, 8)
            pltpu.sync_copy(src_hbm.at[wid, pl.ds(off, _KB)], src_v)
            pltpu.sync_copy(dst_hbm.at[wid, pl.ds(off, _KB)], dst_v)
            gathers = [None] * _NBUF
            scatters = [None] * _NBUF
            for j in range(_NBUF - 1):
                gathers[j] = pltpu.async_copy(
                    x_hbm.at[src_v.at[j]], rows_v.at[j], gsem[j])
            for j in range(_KB):
                p = j % _NBUF
                gathers[p].wait()
                scatters[p] = pltpu.async_copy(
                    rows_v.at[p], acc.at[dst_v.at[j]], ssem[p], add=True)
                nj = j + _NBUF - 1
                if nj < _KB:
                    q = nj % _NBUF
                    if scatters[q] is not None:
                        scatters[q].wait()
                    gathers[q] = pltpu.async_copy(
                        x_hbm.at[src_v.at[nj]], rows_v.at[q], gsem[q])
            for s_ in scatters:
                if s_ is not None:
                    s_.wait()
            return carry

        lax.fori_loop(0, c // _KB, body, 0)
        plsc.subcore_barrier()
        pltpu.sync_copy(
            acc.at[pl.ds(sid * _ZROWS, _ZROWS)],
            out_hbm.at[cid, pl.ds(sid * _ZROWS, _ZROWS)],
        )

    return run(x, src3, dst3, zrows)


_RB = 1000


def _self_body(x, wr, bl, o):
    o[...] = (jnp.dot(x[...], wr[...], preferred_element_type=jnp.float32)
              + bl[...])


def _self_term(x, wr_t, bl):
    return pl.pallas_call(
        _self_body,
        grid=(_N // _RB,),
        in_specs=[
            pl.BlockSpec((_RB, _D), lambda i: (i, 0)),
            pl.BlockSpec((_D, _D), lambda i: (0, 0)),
            pl.BlockSpec((_D,), lambda i: (0,)),
        ],
        out_specs=pl.BlockSpec((_RB, _D), lambda i: (i, 0)),
        out_shape=jax.ShapeDtypeStruct((_N, _D), jnp.float32),
    )(x, wr_t, bl)


def _combine_body(s0, s1, c0, c1, sf, wl, o, *, relu):
    cnt = c0[:, 0:1] + c1[:, 0:1]
    inv = 1.0 / jnp.maximum(cnt, 1.0)
    mean = (s0[...] + s1[...]) * inv
    y = jnp.dot(mean, wl[...], preferred_element_type=jnp.float32) + sf[...]
    o[...] = jnp.maximum(y, 0.0) if relu else y


def _combine(s0, s1, c0, c1, sf, wl_t, relu):
    return pl.pallas_call(
        functools.partial(_combine_body, relu=relu),
        grid=(_N // _RB,),
        in_specs=[
            pl.BlockSpec((_RB, _D), lambda i: (i, 0)),
            pl.BlockSpec((_RB, _D), lambda i: (i, 0)),
            pl.BlockSpec((_RB, _D), lambda i: (i, 0)),
            pl.BlockSpec((_RB, _D), lambda i: (i, 0)),
            pl.BlockSpec((_RB, _D), lambda i: (i, 0)),
            pl.BlockSpec((_D, _D), lambda i: (0, 0)),
        ],
        out_specs=pl.BlockSpec((_RB, _D), lambda i: (i, 0)),
        out_shape=jax.ShapeDtypeStruct((_N, _D), jnp.float32),
    )(s0, s1, c0, c1, sf, wl_t)


def _linear2_body(x, w1, b1, w2, b2, o):
    h = jnp.dot(x[...], w1[...], preferred_element_type=jnp.float32) + b1[...]
    o[...] = jnp.dot(h, w2[...], preferred_element_type=jnp.float32) + b2[...]


def _linear2(x, w1_t, b1, w2_t, b2):
    return pl.pallas_call(
        _linear2_body,
        grid=(_N // _RB,),
        in_specs=[
            pl.BlockSpec((_RB, _D), lambda i: (i, 0)),
            pl.BlockSpec((_D, _D), lambda i: (0, 0)),
            pl.BlockSpec((_D,), lambda i: (0,)),
            pl.BlockSpec((_D, _D), lambda i: (0, 0)),
            pl.BlockSpec((_D,), lambda i: (0,)),
        ],
        out_specs=pl.BlockSpec((_RB, _D), lambda i: (i, 0)),
        out_shape=jax.ShapeDtypeStruct((_N, _D), jnp.float32),
    )(x, w1_t, b1, w2_t, b2)


def kernel(x, edge_index_connections, edge_index_destinations, edge_index_trains,
           conv1_Wl, conv1_bl, conv1_Wr,
           conv2_Wl, conv2_bl, conv2_Wr,
           conv3_Wl, conv3_bl, conv3_Wr,
           conv4_Wl, conv4_bl, conv4_Wr,
           conv5_Wl, conv5_bl, conv5_Wr,
           lin1_W, lin1_b, lin2_W, lin2_b):
    zrows = jnp.zeros((_ZROWS, _D), jnp.float32)
    ones_x = jnp.ones((_N, _D), jnp.float32)

    con = _prep_edges(edge_index_connections)
    dest = _prep_edges(edge_index_destinations)
    trn = _prep_edges(edge_index_trains)

    cnt_con = _sc_segsum(ones_x, con[0], con[1], zrows)
    cnt_dest = _sc_segsum(ones_x, dest[0], dest[1], zrows)
    cnt_trn = _sc_segsum(ones_x, trn[0], trn[1], zrows)

    layers = [
        (con, cnt_con, conv1_Wl, conv1_bl, conv1_Wr, True),
        (trn, cnt_trn, conv2_Wl, conv2_bl, conv2_Wr, False),
        (con, cnt_con, conv3_Wl, conv3_bl, conv3_Wr, True),
        (con, cnt_con, conv3_Wl, conv3_bl, conv3_Wr, True),
        (dest, cnt_dest, conv4_Wl, conv4_bl, conv4_Wr, True),
        (con, cnt_con, conv5_Wl, conv5_bl, conv5_Wr, True),
        (con, cnt_con, conv5_Wl, conv5_bl, conv5_Wr, True),
    ]

    h = x
    for (src3, dst3), cnt, wl, bl, wr, relu in layers:
        p = _sc_segsum(h, src3, dst3, zrows)
        sf = _self_term(h, wr.T, bl)
        h = _combine(p[0], p[1], cnt[0], cnt[1], sf, wl.T, relu)

    return _linear2(h, lin1_W.T, lin1_b, lin2_W.T, lin2_b)


# speedup vs baseline: 13.4401x; 1.0739x over previous
import functools
import math

import jax
import jax.numpy as jnp
from jax import lax
from jax.experimental import pallas as pl
from jax.experimental.pallas import tpu as pltpu
from jax.experimental.pallas import tpu_sc as plsc

_N = 10000
_D = 128
_NC = 2
_NS = 16
_CHUNK = 120
_KB = 16
_NBUF = 3
_ACC_ROWS = 10112
_ZROWS = _ACC_ROWS // _NS


def _prep_edges(ei):
    e = ei.shape[1]
    grp = _NC * _NS * _CHUNK * _KB
    c = _KB * math.ceil(e / grp)
    ep = c * _NC * _NS * _CHUNK
    pad = ep - e
    fill = (jnp.arange(pad, dtype=jnp.int32) * 37) % _N
    src = jnp.concatenate([ei[0], fill])
    dst = jnp.concatenate([ei[1], jnp.full((pad,), _N, jnp.int32)])
    return (src.reshape(_NC * _NS, c, _CHUNK),
            dst.reshape(_NC * _NS, c, _CHUNK))


def _sc_segsum(x, src3, dst3, zrows):
    c = src3.shape[1]

    @functools.partial(
        pl.kernel,
        out_type=jax.ShapeDtypeStruct((_NC, _ACC_ROWS, _D), jnp.float32),
        mesh=plsc.VectorSubcoreMesh(core_axis_name="c", subcore_axis_name="s"),
        scratch_types=[
            pltpu.VMEM((_KB, _CHUNK), jnp.int32),
            pltpu.VMEM((_KB, _CHUNK), jnp.int32),
            pltpu.VMEM((_NBUF, _CHUNK, _D), jnp.float32),
            pltpu.VMEM_SHARED((_ACC_ROWS, _D), jnp.float32),
            [pltpu.SemaphoreType.DMA] * _NBUF,
            [pltpu.SemaphoreType.DMA] * _NBUF,
        ],
    )
    def run(x_hbm, src_hbm, dst_hbm, z_hbm, out_hbm, src_v, dst_v, rows_v, acc,
            gsem, ssem):
        cid = lax.axis_index("c")
        sid = lax.axis_index("s")
        wid = cid * _NS + sid
        pltpu.sync_copy(z_hbm, acc.at[pl.ds(sid * _ZROWS, _ZROWS)])
        plsc.subcore_barrier()

        def body(b, carry):
            off = pl.multiple_of(b * _KB, 8)
            pltpu.sync_copy(src_hbm.at[wid, pl.ds(off, _KB)], src_v)
            pltpu.sync_copy(dst_hbm.at[wid, pl.ds(off, _KB)], dst_v)
            gathers = [None] * _NBUF
            scatters = [None] * _NBUF
            for j in range(_NBUF - 1):
                gathers[j] = pltpu.async_copy(
                    x_hbm.at[src_v.at[j]], rows_v.at[j], gsem[j])
            for j in range(_KB):
                p = j % _NBUF
                gathers[p].wait()
                scatters[p] = pltpu.async_copy(
                    rows_v.at[p], acc.at[dst_v.at[j]], ssem[p], add=True)
                nj = j + _NBUF - 1
                if nj < _KB:
                    q = nj % _NBUF
                    if scatters[q] is not None:
                        scatters[q].wait()
                    gathers[q] = pltpu.async_copy(
                        x_hbm.at[src_v.at[nj]], rows_v.at[q], gsem[q])
            for s_ in scatters:
                if s_ is not None:
                    s_.wait()
            return carry

        lax.fori_loop(0, c // _KB, body, 0)
        plsc.subcore_barrier()
        pltpu.sync_copy(
            acc.at[pl.ds(sid * _ZROWS, _ZROWS)],
            out_hbm.at[cid, pl.ds(sid * _ZROWS, _ZROWS)],
        )

    return run(x, src3, dst3, zrows)


def _sc_count(dst3, ones_rows, zrows):
    c = dst3.shape[1]

    @functools.partial(
        pl.kernel,
        out_type=jax.ShapeDtypeStruct((_NC, _ACC_ROWS, _D), jnp.float32),
        mesh=plsc.VectorSubcoreMesh(core_axis_name="c", subcore_axis_name="s"),
        scratch_types=[
            pltpu.VMEM((_KB, _CHUNK), jnp.int32),
            pltpu.VMEM((_CHUNK, _D), jnp.float32),
            pltpu.VMEM_SHARED((_ACC_ROWS, _D), jnp.float32),
            [pltpu.SemaphoreType.DMA] * _NBUF,
        ],
    )
    def run(ones_hbm, dst_hbm, z_hbm, out_hbm, dst_v, rows_v, acc, ssem):
        cid = lax.axis_index("c")
        sid = lax.axis_index("s")
        wid = cid * _NS + sid
        pltpu.sync_copy(z_hbm, acc.at[pl.ds(sid * _ZROWS, _ZROWS)])
        pltpu.sync_copy(ones_hbm, rows_v)
        plsc.subcore_barrier()

        def body(b, carry):
            off = pl.multiple_of(b * _KB, 8)
            pltpu.sync_copy(dst_hbm.at[wid, pl.ds(off, _KB)], dst_v)
            scatters = [None] * _NBUF
            for j in range(_KB):
                p = j % _NBUF
                if scatters[p] is not None:
                    scatters[p].wait()
                scatters[p] = pltpu.async_copy(
                    rows_v, acc.at[dst_v.at[j]], ssem[p], add=True)
            for s_ in scatters:
                if s_ is not None:
                    s_.wait()
            return carry

        lax.fori_loop(0, c // _KB, body, 0)
        plsc.subcore_barrier()
        pltpu.sync_copy(
            acc.at[pl.ds(sid * _ZROWS, _ZROWS)],
            out_hbm.at[cid, pl.ds(sid * _ZROWS, _ZROWS)],
        )

    return run(ones_rows, dst3, zrows)


_RB = 1000


def _self_body(x, wr, bl, o):
    o[...] = (jnp.dot(x[...], wr[...], preferred_element_type=jnp.float32)
              + bl[...])


def _self_term(x, wr_t, bl):
    return pl.pallas_call(
        _self_body,
        grid=(_N // _RB,),
        in_specs=[
            pl.BlockSpec((_RB, _D), lambda i: (i, 0)),
            pl.BlockSpec((_D, _D), lambda i: (0, 0)),
            pl.BlockSpec((_D,), lambda i: (0,)),
        ],
        out_specs=pl.BlockSpec((_RB, _D), lambda i: (i, 0)),
        out_shape=jax.ShapeDtypeStruct((_N, _D), jnp.float32),
    )(x, wr_t, bl)


def _combine_body(s0, s1, c0, c1, sf, wl, o, *, relu):
    cnt = c0[:, 0:1] + c1[:, 0:1]
    inv = 1.0 / jnp.maximum(cnt, 1.0)
    mean = (s0[...] + s1[...]) * inv
    y = jnp.dot(mean, wl[...], preferred_element_type=jnp.float32) + sf[...]
    o[...] = jnp.maximum(y, 0.0) if relu else y


def _combine(s0, s1, c0, c1, sf, wl_t, relu):
    return pl.pallas_call(
        functools.partial(_combine_body, relu=relu),
        grid=(_N // _RB,),
        in_specs=[
            pl.BlockSpec((_RB, _D), lambda i: (i, 0)),
            pl.BlockSpec((_RB, _D), lambda i: (i, 0)),
            pl.BlockSpec((_RB, _D), lambda i: (i, 0)),
            pl.BlockSpec((_RB, _D), lambda i: (i, 0)),
            pl.BlockSpec((_RB, _D), lambda i: (i, 0)),
            pl.BlockSpec((_D, _D), lambda i: (0, 0)),
        ],
        out_specs=pl.BlockSpec((_RB, _D), lambda i: (i, 0)),
        out_shape=jax.ShapeDtypeStruct((_N, _D), jnp.float32),
    )(s0, s1, c0, c1, sf, wl_t)


def _linear2_body(x, w1, b1, w2, b2, o):
    h = jnp.dot(x[...], w1[...], preferred_element_type=jnp.float32) + b1[...]
    o[...] = jnp.dot(h, w2[...], preferred_element_type=jnp.float32) + b2[...]


def _linear2(x, w1_t, b1, w2_t, b2):
    return pl.pallas_call(
        _linear2_body,
        grid=(_N // _RB,),
        in_specs=[
            pl.BlockSpec((_RB, _D), lambda i: (i, 0)),
            pl.BlockSpec((_D, _D), lambda i: (0, 0)),
            pl.BlockSpec((_D,), lambda i: (0,)),
            pl.BlockSpec((_D, _D), lambda i: (0, 0)),
            pl.BlockSpec((_D,), lambda i: (0,)),
        ],
        out_specs=pl.BlockSpec((_RB, _D), lambda i: (i, 0)),
        out_shape=jax.ShapeDtypeStruct((_N, _D), jnp.float32),
    )(x, w1_t, b1, w2_t, b2)


def kernel(x, edge_index_connections, edge_index_destinations, edge_index_trains,
           conv1_Wl, conv1_bl, conv1_Wr,
           conv2_Wl, conv2_bl, conv2_Wr,
           conv3_Wl, conv3_bl, conv3_Wr,
           conv4_Wl, conv4_bl, conv4_Wr,
           conv5_Wl, conv5_bl, conv5_Wr,
           lin1_W, lin1_b, lin2_W, lin2_b):
    zrows = jnp.zeros((_ZROWS, _D), jnp.float32)
    ones_rows = jnp.ones((_CHUNK, _D), jnp.float32)

    con = _prep_edges(edge_index_connections)
    dest = _prep_edges(edge_index_destinations)
    trn = _prep_edges(edge_index_trains)

    cnt_con = _sc_count(con[1], ones_rows, zrows)
    cnt_dest = _sc_count(dest[1], ones_rows, zrows)
    cnt_trn = _sc_count(trn[1], ones_rows, zrows)

    layers = [
        (con, cnt_con, conv1_Wl, conv1_bl, conv1_Wr, True),
        (trn, cnt_trn, conv2_Wl, conv2_bl, conv2_Wr, False),
        (con, cnt_con, conv3_Wl, conv3_bl, conv3_Wr, True),
        (con, cnt_con, conv3_Wl, conv3_bl, conv3_Wr, True),
        (dest, cnt_dest, conv4_Wl, conv4_bl, conv4_Wr, True),
        (con, cnt_con, conv5_Wl, conv5_bl, conv5_Wr, True),
        (con, cnt_con, conv5_Wl, conv5_bl, conv5_Wr, True),
    ]

    h = x
    for (src3, dst3), cnt, wl, bl, wr, relu in layers:
        p = _sc_segsum(h, src3, dst3, zrows)
        sf = _self_term(h, wr.T, bl)
        h = _combine(p[0], p[1], cnt[0], cnt[1], sf, wl.T, relu)

    return _linear2(h, lin1_W.T, lin1_b, lin2_W.T, lin2_b)
